# Initial kernel scaffold; baseline (speedup 1.0000x reference)
import jax
import jax.numpy as jnp
from jax.experimental import pallas as pl


def kernel(x, table):
    raise NotImplementedError("write your pallas kernel here")


# speedup vs baseline: 9.5278x; 9.5278x over previous
import functools

import jax
import jax.numpy as jnp
from jax import lax
from jax.experimental import pallas as pl
from jax.experimental.pallas import tpu as pltpu
from jax.experimental.pallas import tpu_sc as plsc

D = 128
NC, NS = 2, 16
NW = NC * NS
BATCH, HIST = 4096, 200
B = BATCH * HIST
B_PER_W = B // NW
CHUNK = 128
NCH = B_PER_W // CHUNK

_mesh = plsc.VectorSubcoreMesh(core_axis_name="c", subcore_axis_name="s")


@functools.partial(
    pl.kernel,
    mesh=_mesh,
    out_type=jax.ShapeDtypeStruct((B, D), jnp.float32),
    scratch_types=[
        pltpu.VMEM((NCH, CHUNK), jnp.int32),
        pltpu.VMEM((CHUNK, D), jnp.float32),
        pltpu.VMEM((CHUNK, D), jnp.float32),
        pltpu.SemaphoreType.DMA,
        pltpu.SemaphoreType.DMA,
    ],
)
def _emb_lookup(x_hbm, table_hbm, out_hbm, idx_v, rows0, rows1, sem0, sem1):
    wid = lax.axis_index("s") * NC + lax.axis_index("c")
    base = wid * B_PER_W

    pltpu.sync_copy(x_hbm.at[pl.ds(wid * NCH, NCH)], idx_v)

    pltpu.async_copy(table_hbm.at[idx_v.at[0]], rows0, sem0)
    pltpu.async_copy(table_hbm.at[idx_v.at[1]], rows1, sem1)

    def body(g, carry):
        c0 = 2 * g

        pltpu.make_async_copy(table_hbm.at[idx_v.at[0]], rows0, sem0).wait()
        pltpu.sync_copy(rows0, out_hbm.at[pl.ds(base + c0 * CHUNK, CHUNK)])

        @pl.when(g < NCH // 2 - 1)
        def _():
            pltpu.async_copy(table_hbm.at[idx_v.at[c0 + 2]], rows0, sem0)

        pltpu.make_async_copy(table_hbm.at[idx_v.at[1]], rows1, sem1).wait()
        pltpu.sync_copy(rows1, out_hbm.at[pl.ds(base + (c0 + 1) * CHUNK, CHUNK)])

        @pl.when(g < NCH // 2 - 1)
        def _():
            pltpu.async_copy(table_hbm.at[idx_v.at[c0 + 3]], rows1, sem1)

        return carry

    lax.fori_loop(0, NCH // 2, body, None)


def kernel(x, table):
    x2 = x.reshape(NW * NCH, CHUNK).astype(jnp.int32)
    out = _emb_lookup(x2, table)
    return out.reshape(BATCH, HIST, D)


# speedup vs baseline: 9.6886x; 1.0169x over previous
import functools

import jax
import jax.numpy as jnp
from jax import lax
from jax.experimental import pallas as pl
from jax.experimental.pallas import tpu as pltpu
from jax.experimental.pallas import tpu_sc as plsc

D = 128
NC, NS = 2, 16
NW = NC * NS
BATCH, HIST = 4096, 200
B = BATCH * HIST
B_PER_W = B // NW
GROW = 128
GPC = 2
CHUNK = GROW * GPC
NCH = B_PER_W // CHUNK
NIR = B_PER_W // GROW

_mesh = plsc.VectorSubcoreMesh(core_axis_name="c", subcore_axis_name="s")


@functools.partial(
    pl.kernel,
    mesh=_mesh,
    out_type=jax.ShapeDtypeStruct((B, D), jnp.float32),
    scratch_types=[
        pltpu.VMEM((NIR, GROW), jnp.int32),
        pltpu.VMEM((CHUNK, D), jnp.float32),
        pltpu.VMEM((CHUNK, D), jnp.float32),
        pltpu.SemaphoreType.DMA,
        pltpu.SemaphoreType.DMA,
    ],
)
def _emb_lookup(x_hbm, table_hbm, out_hbm, idx_v, rows0, rows1, sem0, sem1):
    wid = lax.axis_index("s") * NC + lax.axis_index("c")
    base = wid * B_PER_W

    pltpu.sync_copy(x_hbm.at[pl.ds(wid * NIR, NIR)], idx_v)

    def fire(c, rows, sem):
        for j in range(GPC):
            pltpu.async_copy(
                table_hbm.at[idx_v.at[GPC * c + j]],
                rows.at[pl.ds(j * GROW, GROW)],
                sem,
            )

    def drain(rows, sem):
        for j in range(GPC):
            pltpu.make_async_copy(
                table_hbm.at[idx_v.at[j]],
                rows.at[pl.ds(j * GROW, GROW)],
                sem,
            ).wait()

    fire(0, rows0, sem0)
    fire(1, rows1, sem1)

    def body(g, carry):
        c0 = 2 * g

        drain(rows0, sem0)
        pltpu.sync_copy(rows0, out_hbm.at[pl.ds(base + c0 * CHUNK, CHUNK)])

        @pl.when(g < NCH // 2 - 1)
        def _():
            fire(c0 + 2, rows0, sem0)

        drain(rows1, sem1)
        pltpu.sync_copy(rows1, out_hbm.at[pl.ds(base + (c0 + 1) * CHUNK, CHUNK)])

        @pl.when(g < NCH // 2 - 1)
        def _():
            fire(c0 + 3, rows1, sem1)

        return carry

    lax.fori_loop(0, NCH // 2, body, None)


def kernel(x, table):
    x2 = x.reshape(NW * NIR, GROW).astype(jnp.int32)
    out = _emb_lookup(x2, table)
    return out.reshape(BATCH, HIST, D)
